# baseline (speedup 1.0000x reference)
import functools

import jax
import jax.numpy as jnp
from jax import lax
from jax.experimental import pallas as pl
from jax.experimental.pallas import tpu as pltpu
from jax.experimental.pallas import tpu_sc as plsc

N_NODES = 10000
N_PAD = 10112
D = 128
DW = 144
N_EDGES = 320000
NC = 2
NS = 16
NW = NC * NS
CH = 128
NE_PAD = 327680
ROWS_E = NE_PAD // CH
K = ROWS_E // NW
KB = 8
NODES_PER_SUB = N_PAD // NS

ALPHA = 0.8
BETA = 1.0 - ALPHA

_mesh = plsc.VectorSubcoreMesh(core_axis_name="c", subcore_axis_name="s")


@functools.partial(
    pl.kernel,
    out_type=[
        jax.ShapeDtypeStruct((NC, N_PAD, DW), jnp.float32),
    ],
    mesh=_mesh,
    compiler_params=pltpu.CompilerParams(use_tc_tiling_on_sc=False),
    scratch_types=[
        pltpu.VMEM_SHARED((N_PAD, DW), jnp.float32),
        pltpu.VMEM((KB * 3, CH), jnp.int32),
        pltpu.VMEM((CH, DW), jnp.float32),
        pltpu.VMEM((CH, DW), jnp.float32),
        pltpu.SemaphoreType.DMA,
        pltpu.SemaphoreType.DMA,
        pltpu.SemaphoreType.DMA,
        pltpu.SemaphoreType.DMA,
    ],
)
def _sc_scatter(e_hbm, feats_hbm, pm_hbm,
                acc_m, ev, rows, rows1,
                gsem, gsem1, ssem, ssem1):
    c = lax.axis_index("c")
    s = lax.axis_index("s")
    wid = c * NS + s
    zero16 = jnp.zeros((16,), jnp.float32)

    for i in range(CH):
        for dcol in range(DW // 16):
            rows[i, pl.ds(dcol * 16, 16)] = zero16

    def _zero_m(i, carry):
        pltpu.sync_copy(rows, acc_m.at[pl.ds(s * NODES_PER_SUB + i * CH, CH)])
        return carry
    lax.fori_loop(0, NODES_PER_SUB // CH, _zero_m, 0)
    tail = NODES_PER_SUB % CH
    pltpu.sync_copy(
        rows.at[pl.ds(0, tail)],
        acc_m.at[pl.ds(s * NODES_PER_SUB + NODES_PER_SUB - tail, tail)])

    plsc.subcore_barrier()

    iota16 = lax.iota(jnp.int32, 16)

    bufs = (rows, rows1)
    gsems = (gsem, gsem1)
    ssems = (ssem, ssem1)

    def _block(bk, carry0):
        pltpu.sync_copy(e_hbm.at[pl.ds((wid * K + bk * KB) * 3, KB * 3)], ev)

        gd = [None] * KB
        sd = [None] * KB
        gd[0] = pltpu.async_copy(feats_hbm.at[ev.at[0]], bufs[0], gsems[0])
        for i in range(KB):
            rb = bufs[i % 2]
            if i + 1 < KB:
                if i >= 1:
                    sd[i - 1].wait()
                gd[i + 1] = pltpu.async_copy(
                    feats_hbm.at[ev.at[3 * (i + 1)]], bufs[(i + 1) % 2],
                    gsems[(i + 1) % 2])
            gd[i].wait()

            def _group(g, carry2, rb=rb, i=i):
                w16 = lax.bitcast_convert_type(
                    ev[3 * i + 2, pl.ds(g * 16, 16)], jnp.float32)
                for l in range(16):
                    e = g * 16 + l
                    wsc = w16[l]
                    for dcol in range(DW // 16):
                        rb[e, pl.ds(dcol * 16, 16)] = (
                            rb[e, pl.ds(dcol * 16, 16)] * wsc)
                return carry2
            lax.fori_loop(0, CH // 16, _group, 0)

            sd[i] = pltpu.async_copy(rb, acc_m.at[ev.at[3 * i + 1]],
                                     ssems[i % 2], add=True)
        sd[KB - 2].wait()
        sd[KB - 1].wait()
        return carry0
    lax.fori_loop(0, K // KB, _block, 0)

    plsc.subcore_barrier()

    base = s * NODES_PER_SUB
    pltpu.sync_copy(acc_m.at[pl.ds(base, NODES_PER_SUB)],
                    pm_hbm.at[c, pl.ds(base, NODES_PER_SUB)])


def _tc_body(pm_ref, feats_ref, w_ref, b_ref, out_ref):
    psum = pm_ref[0] + pm_ref[1]
    sum_m = psum[:, :D]
    sum_w = psum[:, D:D + 1]
    pos = sum_w > 0.0
    h_neigh = jnp.where(pos, sum_m / jnp.where(pos, sum_w, 1.0), 0.0)
    agg = ALPHA * feats_ref[...] + BETA * h_neigh
    z = lax.dot_general(agg, w_ref[...], (((1,), (1,)), ((), ())),
                        preferred_element_type=jnp.float32)
    z = z + b_ref[...]
    out_ref[...] = 1.0 / (1.0 + jnp.exp(-z))


_TB = 1000

_tc_call = pl.pallas_call(
    _tc_body,
    grid=(N_NODES // _TB,),
    in_specs=[
        pl.BlockSpec((NC, _TB, DW), lambda i: (0, i, 0)),
        pl.BlockSpec((_TB, D), lambda i: (i, 0)),
        pl.BlockSpec((D, D), lambda i: (0, 0)),
        pl.BlockSpec((1, D), lambda i: (0, 0)),
    ],
    out_specs=pl.BlockSpec((_TB, D), lambda i: (i, 0)),
    out_shape=jax.ShapeDtypeStruct((N_NODES, D), jnp.float32),
)


def kernel(feats, edge_index, edge_weight, W, b):
    src = edge_index[0]
    dst = edge_index[1]
    pad = NE_PAD - N_EDGES
    pad_idx = jnp.arange(pad, dtype=jnp.int32) % N_NODES
    src_p = jnp.concatenate([src, pad_idx]).reshape(ROWS_E, CH)
    dst_p = jnp.concatenate([dst, pad_idx]).reshape(ROWS_E, CH)
    w_p = jnp.concatenate(
        [edge_weight, jnp.zeros((pad,), jnp.float32)]).reshape(ROWS_E, CH)
    w_bits = lax.bitcast_convert_type(w_p, jnp.int32)
    e_pack = jnp.stack([src_p, dst_p, w_bits], axis=1).reshape(ROWS_E * 3, CH)
    feats_ext = jnp.concatenate(
        [feats, jnp.ones((N_NODES, 1), jnp.float32),
         jnp.zeros((N_NODES, DW - D - 1), jnp.float32)], axis=1)
    (pm,) = _sc_scatter(e_pack, feats_ext)
    return _tc_call(pm, feats, W, b.reshape(1, D))


# speedup vs baseline: 1.0845x; 1.0845x over previous
import functools

import jax
import jax.numpy as jnp
from jax import lax
from jax.experimental import pallas as pl
from jax.experimental.pallas import tpu as pltpu
from jax.experimental.pallas import tpu_sc as plsc

N_NODES = 10000
N_PAD = 10240
D = 128
N_EDGES = 320000
NC = 2
NS = 16
NW = NC * NS
CH = 128
NE_PAD = 327680
ROWS_E = NE_PAD // CH
K = ROWS_E // NW
KB = 8
NODES_PER_SUB = N_PAD // NS

ALPHA = 0.8
BETA = 1.0 - ALPHA

_mesh = plsc.VectorSubcoreMesh(core_axis_name="c", subcore_axis_name="s")


@functools.partial(
    pl.kernel,
    out_type=[
        jax.ShapeDtypeStruct((NC, N_PAD, D), jnp.float32),
        jax.ShapeDtypeStruct((NC, N_PAD, 16), jnp.float32),
    ],
    mesh=_mesh,
    compiler_params=pltpu.CompilerParams(use_tc_tiling_on_sc=False),
    scratch_types=[
        pltpu.VMEM_SHARED((N_PAD, D), jnp.float32),
        pltpu.VMEM_SHARED((N_PAD, 16), jnp.float32),
        pltpu.VMEM((KB * 3, CH), jnp.int32),
        pltpu.VMEM((CH, D), jnp.float32),
        pltpu.VMEM((CH, D), jnp.float32),
        pltpu.VMEM((CH, 16), jnp.float32),
        pltpu.SemaphoreType.DMA,
        pltpu.SemaphoreType.DMA,
        pltpu.SemaphoreType.DMA,
        pltpu.SemaphoreType.DMA,
    ],
)
def _sc_scatter(e_hbm, feats_hbm, pm_hbm, pw_hbm,
                acc_m, acc_w, ev, rows, rows1, wrow,
                gsem, gsem1, ssem, ssem1):
    c = lax.axis_index("c")
    s = lax.axis_index("s")
    wid = c * NS + s
    zero16 = jnp.zeros((16,), jnp.float32)

    for i in range(CH):
        for dcol in range(D // 16):
            rows[i, pl.ds(dcol * 16, 16)] = zero16
    for e in range(CH):
        wrow[e, :] = zero16

    def _zero_m(i, carry):
        pltpu.sync_copy(rows, acc_m.at[pl.ds(s * NODES_PER_SUB + i * CH, CH)])
        pltpu.sync_copy(wrow,
                        acc_w.at[pl.ds(s * NODES_PER_SUB + i * CH, CH)])
        return carry
    lax.fori_loop(0, NODES_PER_SUB // CH, _zero_m, 0)

    plsc.subcore_barrier()

    iota16 = lax.iota(jnp.int32, 16)

    bufs = (rows, rows1)
    gsems = (gsem, gsem1)
    ssems = (ssem, ssem1)

    def _block(bk, carry0):
        pltpu.sync_copy(e_hbm.at[pl.ds((wid * K + bk * KB) * 3, KB * 3)], ev)

        gd = [None] * KB
        sd = [None] * KB
        gd[0] = pltpu.async_copy(feats_hbm.at[ev.at[0]], bufs[0], gsems[0])
        for i in range(KB):
            rb = bufs[i % 2]
            if i + 1 < KB:
                if i >= 1:
                    sd[i - 1].wait()
                gd[i + 1] = pltpu.async_copy(
                    feats_hbm.at[ev.at[3 * (i + 1)]], bufs[(i + 1) % 2],
                    gsems[(i + 1) % 2])
            gd[i].wait()

            @plsc.parallel_loop(0, CH // 16)
            def _group(g, rb=rb, i=i):
                w16 = lax.bitcast_convert_type(
                    ev[3 * i + 2, pl.ds(g * 16, 16)], jnp.float32)
                for l in range(16):
                    e = g * 16 + l
                    wsc = w16[l]
                    wrow[e, :] = jnp.where(iota16 == 0, wsc, 0.0)
                    for dcol in range(D // 16):
                        rb[e, pl.ds(dcol * 16, 16)] = (
                            rb[e, pl.ds(dcol * 16, 16)] * wsc)

            sd[i] = pltpu.async_copy(rb, acc_m.at[ev.at[3 * i + 1]],
                                     ssems[i % 2], add=True)
            pltpu.sync_copy(wrow, acc_w.at[ev.at[3 * i + 1]], add=True)
        sd[KB - 2].wait()
        sd[KB - 1].wait()
        return carry0
    lax.fori_loop(0, K // KB, _block, 0)

    plsc.subcore_barrier()

    base = s * NODES_PER_SUB
    pltpu.sync_copy(acc_m.at[pl.ds(base, NODES_PER_SUB)],
                    pm_hbm.at[c, pl.ds(base, NODES_PER_SUB)])
    pltpu.sync_copy(acc_w.at[pl.ds(base, NODES_PER_SUB)],
                    pw_hbm.at[c, pl.ds(base, NODES_PER_SUB)])


def _tc_body(pm_ref, pw_ref, feats_ref, w_ref, b_ref, out_ref):
    sum_m = pm_ref[0] + pm_ref[1]
    sum_w = pw_ref[0][:, 0:1] + pw_ref[1][:, 0:1]
    pos = sum_w > 0.0
    h_neigh = jnp.where(pos, sum_m / jnp.where(pos, sum_w, 1.0), 0.0)
    agg = ALPHA * feats_ref[...] + BETA * h_neigh
    z = lax.dot_general(agg, w_ref[...], (((1,), (1,)), ((), ())),
                        preferred_element_type=jnp.float32)
    z = z + b_ref[...]
    out_ref[...] = 1.0 / (1.0 + jnp.exp(-z))


_TB = 1000

_tc_call = pl.pallas_call(
    _tc_body,
    grid=(N_NODES // _TB,),
    in_specs=[
        pl.BlockSpec((NC, _TB, D), lambda i: (0, i, 0)),
        pl.BlockSpec((NC, _TB, 16), lambda i: (0, i, 0)),
        pl.BlockSpec((_TB, D), lambda i: (i, 0)),
        pl.BlockSpec((D, D), lambda i: (0, 0)),
        pl.BlockSpec((1, D), lambda i: (0, 0)),
    ],
    out_specs=pl.BlockSpec((_TB, D), lambda i: (i, 0)),
    out_shape=jax.ShapeDtypeStruct((N_NODES, D), jnp.float32),
)


def kernel(feats, edge_index, edge_weight, W, b):
    src = edge_index[0]
    dst = edge_index[1]
    pad = NE_PAD - N_EDGES
    pad_idx = jnp.arange(pad, dtype=jnp.int32) % N_NODES
    src_p = jnp.concatenate([src, pad_idx]).reshape(ROWS_E, CH)
    dst_p = jnp.concatenate([dst, pad_idx]).reshape(ROWS_E, CH)
    w_p = jnp.concatenate(
        [edge_weight, jnp.zeros((pad,), jnp.float32)]).reshape(ROWS_E, CH)
    w_bits = lax.bitcast_convert_type(w_p, jnp.int32)
    e_pack = jnp.stack([src_p, dst_p, w_bits], axis=1).reshape(ROWS_E * 3, CH)
    pm, pw = _sc_scatter(e_pack, feats)
    return _tc_call(pm, pw, feats, W, b.reshape(1, D))


# speedup vs baseline: 1.0926x; 1.0074x over previous
import functools

import jax
import jax.numpy as jnp
from jax import lax
from jax.experimental import pallas as pl
from jax.experimental.pallas import tpu as pltpu
from jax.experimental.pallas import tpu_sc as plsc

N_NODES = 10000
N_PAD = 10240
D = 128
N_EDGES = 320000
NC = 2
NS = 16
NW = NC * NS
CH = 128
NE_PAD = 327680
ROWS_E = NE_PAD // CH
K = ROWS_E // NW
KB = 8
NODES_PER_SUB = N_PAD // NS

ALPHA = 0.8
BETA = 1.0 - ALPHA

_mesh = plsc.VectorSubcoreMesh(core_axis_name="c", subcore_axis_name="s")


@functools.partial(
    pl.kernel,
    out_type=[
        jax.ShapeDtypeStruct((NC, N_PAD, D), jnp.float32),
        jax.ShapeDtypeStruct((NC, N_PAD, 16), jnp.float32),
    ],
    mesh=_mesh,
    compiler_params=pltpu.CompilerParams(use_tc_tiling_on_sc=False),
    scratch_types=[
        pltpu.VMEM_SHARED((N_PAD, D), jnp.float32),
        pltpu.VMEM_SHARED((N_PAD, 16), jnp.float32),
        pltpu.VMEM((KB * 3, CH), jnp.int32),
        pltpu.VMEM((CH, D), jnp.float32),
        pltpu.VMEM((CH, D), jnp.float32),
        pltpu.VMEM((CH, 16), jnp.float32),
        pltpu.SemaphoreType.DMA,
        pltpu.SemaphoreType.DMA,
        pltpu.SemaphoreType.DMA,
        pltpu.SemaphoreType.DMA,
    ],
)
def _sc_scatter(e_hbm, feats_hbm, pm_hbm, pw_hbm,
                acc_m, acc_w, ev, rows, rows1, wrow,
                gsem, gsem1, ssem, ssem1):
    c = lax.axis_index("c")
    s = lax.axis_index("s")
    wid = c * NS + s
    zero16 = jnp.zeros((16,), jnp.float32)

    for i in range(CH):
        for dcol in range(D // 16):
            rows[i, pl.ds(dcol * 16, 16)] = zero16
    for e in range(CH):
        wrow[e, :] = zero16

    def _zero_m(i, carry):
        pltpu.sync_copy(rows, acc_m.at[pl.ds(s * NODES_PER_SUB + i * CH, CH)])
        pltpu.sync_copy(wrow,
                        acc_w.at[pl.ds(s * NODES_PER_SUB + i * CH, CH)])
        return carry
    lax.fori_loop(0, NODES_PER_SUB // CH, _zero_m, 0)

    plsc.subcore_barrier()

    iota16 = lax.iota(jnp.int32, 16)

    bufs = (rows, rows1)
    gsems = (gsem, gsem1)
    ssems = (ssem, ssem1)

    def _block(bk, carry0):
        pltpu.sync_copy(e_hbm.at[pl.ds((wid * K + bk * KB) * 3, KB * 3)], ev)

        gd = [None] * KB
        sd = [None] * KB
        gd[0] = pltpu.async_copy(feats_hbm.at[ev.at[0]], bufs[0], gsems[0])
        for i in range(KB):
            rb = bufs[i % 2]
            if i + 1 < KB:
                if i >= 1:
                    sd[i - 1].wait()
                gd[i + 1] = pltpu.async_copy(
                    feats_hbm.at[ev.at[3 * (i + 1)]], bufs[(i + 1) % 2],
                    gsems[(i + 1) % 2])
            gd[i].wait()

            @plsc.parallel_loop(0, CH // 16, unroll=2)
            def _group(g, rb=rb, i=i):
                w16 = lax.bitcast_convert_type(
                    ev[3 * i + 2, pl.ds(g * 16, 16)], jnp.float32)
                for l in range(16):
                    e = g * 16 + l
                    wsc = w16[l]
                    wrow[e, :] = jnp.where(iota16 == 0, wsc, 0.0)
                    for dcol in range(D // 16):
                        rb[e, pl.ds(dcol * 16, 16)] = (
                            rb[e, pl.ds(dcol * 16, 16)] * wsc)

            sd[i] = pltpu.async_copy(rb, acc_m.at[ev.at[3 * i + 1]],
                                     ssems[i % 2], add=True)
            pltpu.sync_copy(wrow, acc_w.at[ev.at[3 * i + 1]], add=True)
        sd[KB - 2].wait()
        sd[KB - 1].wait()
        return carry0
    lax.fori_loop(0, K // KB, _block, 0)

    plsc.subcore_barrier()

    base = s * NODES_PER_SUB
    pltpu.sync_copy(acc_m.at[pl.ds(base, NODES_PER_SUB)],
                    pm_hbm.at[c, pl.ds(base, NODES_PER_SUB)])
    pltpu.sync_copy(acc_w.at[pl.ds(base, NODES_PER_SUB)],
                    pw_hbm.at[c, pl.ds(base, NODES_PER_SUB)])


def _tc_body(pm_ref, pw_ref, feats_ref, w_ref, b_ref, out_ref):
    sum_m = pm_ref[0] + pm_ref[1]
    sum_w = pw_ref[0][:, 0:1] + pw_ref[1][:, 0:1]
    pos = sum_w > 0.0
    h_neigh = jnp.where(pos, sum_m / jnp.where(pos, sum_w, 1.0), 0.0)
    agg = ALPHA * feats_ref[...] + BETA * h_neigh
    z = lax.dot_general(agg, w_ref[...], (((1,), (1,)), ((), ())),
                        preferred_element_type=jnp.float32)
    z = z + b_ref[...]
    out_ref[...] = 1.0 / (1.0 + jnp.exp(-z))


_TB = 1000

_tc_call = pl.pallas_call(
    _tc_body,
    grid=(N_NODES // _TB,),
    in_specs=[
        pl.BlockSpec((NC, _TB, D), lambda i: (0, i, 0)),
        pl.BlockSpec((NC, _TB, 16), lambda i: (0, i, 0)),
        pl.BlockSpec((_TB, D), lambda i: (i, 0)),
        pl.BlockSpec((D, D), lambda i: (0, 0)),
        pl.BlockSpec((1, D), lambda i: (0, 0)),
    ],
    out_specs=pl.BlockSpec((_TB, D), lambda i: (i, 0)),
    out_shape=jax.ShapeDtypeStruct((N_NODES, D), jnp.float32),
)


def kernel(feats, edge_index, edge_weight, W, b):
    src = edge_index[0]
    dst = edge_index[1]
    pad = NE_PAD - N_EDGES
    pad_idx = jnp.arange(pad, dtype=jnp.int32) % N_NODES
    src_p = jnp.concatenate([src, pad_idx]).reshape(ROWS_E, CH)
    dst_p = jnp.concatenate([dst, pad_idx]).reshape(ROWS_E, CH)
    w_p = jnp.concatenate(
        [edge_weight, jnp.zeros((pad,), jnp.float32)]).reshape(ROWS_E, CH)
    w_bits = lax.bitcast_convert_type(w_p, jnp.int32)
    e_pack = jnp.stack([src_p, dst_p, w_bits], axis=1).reshape(ROWS_E * 3, CH)
    pm, pw = _sc_scatter(e_pack, feats)
    return _tc_call(pm, pw, feats, W, b.reshape(1, D))
